# baseline (speedup 1.0000x reference)
import functools

import jax
import jax.numpy as jnp
from jax import lax
from jax.experimental import pallas as pl
from jax.experimental.pallas import tpu as pltpu
from jax.experimental.pallas import tpu_sc as plsc

N = 10000
DEG = 32
D = 128

P_TC = 5120
BLK_TC = 512
L = 16
NB = 8
NW = 32
N_SC = N - P_TC
NBLK = N_SC // NB
TMAX = (NBLK + NW - 1) // NW
TMAX += TMAX % 2
NDC = D // L

_mesh = plsc.VectorSubcoreMesh(core_axis_name="c", subcore_axis_name="s")


def _gat_sc_body(a1_hbm, a2_hbm, ft_hbm, out_hbm,
                 ft_buf, a2_buf, a1_buf, out_buf,
                 in_sem0, in_sem1, out_sem0, out_sem1):
    wid = lax.axis_index("c") * 16 + lax.axis_index("s")
    in_sems = (in_sem0, in_sem1)
    out_sems = (out_sem0, out_sem1)

    def in_copies(t, b):
        base = P_TC + (t * NW + wid) * NB
        return (
            pltpu.make_async_copy(ft_hbm.at[pl.ds(base, NB)], ft_buf.at[b],
                                  in_sems[b]),
            pltpu.make_async_copy(a2_hbm.at[:, pl.ds(base, NB)], a2_buf.at[b],
                                  in_sems[b]),
            pltpu.make_async_copy(a1_hbm.at[pl.ds(base, NB)], a1_buf.at[b],
                                  in_sems[b]),
        )

    iota0 = lax.iota(jnp.int32, L)
    iota1 = iota0 + L

    def start_in(t, b):
        @pl.when(t * NW + wid < NBLK)
        def _():
            for c in in_copies(t, b):
                c.start()

    def compute_block(b):
        def node_body(i, _):
            iv = jnp.full((L,), i, jnp.int32)
            a1v = plsc.load_gather(a1_buf.at[b], [iv])
            x0 = plsc.load_gather(a2_buf.at[b], [iota0, iv]) + a1v
            x1 = plsc.load_gather(a2_buf.at[b], [iota1, iv]) + a1v
            l0 = jnp.where(x0 > 0, x0, x0 * 0.01)
            l1 = jnp.where(x1 > 0, x1, x1 * 0.01)
            m = jnp.maximum(jnp.max(l0), jnp.max(l1))
            e0 = jnp.exp(l0 - m)
            e1 = jnp.exp(l1 - m)
            sv = jnp.zeros((L,), jnp.float32) + (jnp.sum(e0) + jnp.sum(e1))
            e0 = e0 / sv
            e1 = e1 / sv

            acc = [jnp.zeros((L,), jnp.float32) for _ in range(NDC)]
            for k in range(DEG):
                w = e0[k] if k < L else e1[k - L]
                for dc in range(NDC):
                    acc[dc] = acc[dc] + w * ft_buf[b, i, k, pl.ds(dc * L, L)]
            for dc in range(NDC):
                out_buf[b, i, pl.ds(dc * L, L)] = acc[dc]
            return 0

        lax.fori_loop(0, NB, node_body, 0)

    start_in(0, 0)
    start_in(1, 1)

    def pair_body(tt, _):
        for b in (0, 1):
            t = tt * 2 + b
            blk = t * NW + wid

            @pl.when(blk < NBLK)
            def _():
                base = blk * NB
                for c in in_copies(t, b):
                    c.wait()

                @pl.when(tt >= 1)
                def _():
                    prev = P_TC + ((t - 2) * NW + wid) * NB
                    pltpu.make_async_copy(
                        out_buf.at[b], out_hbm.at[pl.ds(prev, NB)],
                        out_sems[b]).wait()

                compute_block(b)
                pltpu.make_async_copy(
                    out_buf.at[b], out_hbm.at[pl.ds(P_TC + base, NB)],
                    out_sems[b]).start()
                start_in(t + 2, b)
        return 0

    lax.fori_loop(0, TMAX // 2, pair_body, 0)

    for b in (0, 1):
        pltpu.make_async_copy(out_buf.at[b], out_hbm.at[pl.ds(0, NB)],
                              out_sems[b]).wait()


_gat_sc = functools.partial(
    pl.kernel,
    out_type=jax.ShapeDtypeStruct((N, D), jnp.float32),
    mesh=_mesh,
    compiler_params=pltpu.CompilerParams(needs_layout_passes=False,
                                         use_tc_tiling_on_sc=False,
                                         skip_device_barrier=True),
    scratch_types=[
        pltpu.VMEM((2, NB, DEG, D), jnp.float32),
        pltpu.VMEM((2, DEG, NB), jnp.float32),
        pltpu.VMEM((2, NB), jnp.float32),
        pltpu.VMEM((2, NB, D), jnp.float32),
        pltpu.SemaphoreType.DMA,
        pltpu.SemaphoreType.DMA,
        pltpu.SemaphoreType.DMA,
        pltpu.SemaphoreType.DMA,
    ],
)(_gat_sc_body)


def _gat_tc_block(a1_ref, a2_ref, ft_ref, out_ref):
    i = pl.program_id(0)
    a1 = a1_ref[pl.ds(i * BLK_TC, BLK_TC)]
    a2 = a2_ref[...]
    ft = ft_ref[...]
    a = a1[None, :] + a2
    l = jnp.where(a > 0, a, 0.01 * a)
    m = jnp.max(l, axis=0, keepdims=True)
    e = jnp.exp(l - m)
    wk = e / jnp.sum(e, axis=0, keepdims=True)
    w = wk.T
    out_ref[...] = jnp.sum(w[:, :, None] * ft, axis=1)


def _gat_tc(a1f, a2km, ft):
    return pl.pallas_call(
        _gat_tc_block,
        grid=(P_TC // BLK_TC,),
        in_specs=[
            pl.BlockSpec((N,), lambda i: (0,)),
            pl.BlockSpec((DEG, BLK_TC), lambda i: (0, i)),
            pl.BlockSpec((BLK_TC, DEG, D), lambda i: (i, 0, 0)),
        ],
        out_specs=pl.BlockSpec((BLK_TC, D), lambda i: (i, 0)),
        out_shape=jax.ShapeDtypeStruct((P_TC, D), jnp.float32),
    )(a1f, a2km, ft)


@jax.jit
def kernel(a1, a2, ft):
    a1f = a1.reshape(N)
    a2km = a2.reshape(N, DEG).T
    out_sc = _gat_sc(a1f, a2km, ft)
    out_tc = _gat_tc(a1f, a2km, ft)
    return lax.dynamic_update_slice(out_sc, out_tc, (0, 0))


# speedup vs baseline: 1.0203x; 1.0203x over previous
import functools

import jax
import jax.numpy as jnp
from jax import lax
from jax.experimental import pallas as pl
from jax.experimental.pallas import tpu as pltpu
from jax.experimental.pallas import tpu_sc as plsc

N = 10000
DEG = 32
D = 128

P_TC = 5632
BLK_TC = 512
L = 16
NB = 8
NW = 32
N_SC = N - P_TC
NBLK = N_SC // NB
TMAX = (NBLK + NW - 1) // NW
TMAX += TMAX % 2
NDC = D // L

_mesh = plsc.VectorSubcoreMesh(core_axis_name="c", subcore_axis_name="s")


def _gat_sc_body(a1_hbm, a2_hbm, ft_hbm, out_hbm,
                 ft_buf, a2_buf, a1_buf, out_buf,
                 in_sem0, in_sem1, out_sem0, out_sem1):
    wid = lax.axis_index("c") * 16 + lax.axis_index("s")
    in_sems = (in_sem0, in_sem1)
    out_sems = (out_sem0, out_sem1)

    def in_copies(t, b):
        base = P_TC + (t * NW + wid) * NB
        return (
            pltpu.make_async_copy(ft_hbm.at[pl.ds(base, NB)], ft_buf.at[b],
                                  in_sems[b]),
            pltpu.make_async_copy(a2_hbm.at[:, pl.ds(base, NB)], a2_buf.at[b],
                                  in_sems[b]),
            pltpu.make_async_copy(a1_hbm.at[pl.ds(base, NB)], a1_buf.at[b],
                                  in_sems[b]),
        )

    iota0 = lax.iota(jnp.int32, L)
    iota1 = iota0 + L

    def start_in(t, b):
        @pl.when(t * NW + wid < NBLK)
        def _():
            for c in in_copies(t, b):
                c.start()

    def compute_block(b):
        def node_body(i, _):
            iv = jnp.full((L,), i, jnp.int32)
            a1v = plsc.load_gather(a1_buf.at[b], [iv])
            x0 = plsc.load_gather(a2_buf.at[b], [iota0, iv]) + a1v
            x1 = plsc.load_gather(a2_buf.at[b], [iota1, iv]) + a1v
            l0 = jnp.where(x0 > 0, x0, x0 * 0.01)
            l1 = jnp.where(x1 > 0, x1, x1 * 0.01)
            m = jnp.maximum(jnp.max(l0), jnp.max(l1))
            e0 = jnp.exp(l0 - m)
            e1 = jnp.exp(l1 - m)
            sv = jnp.zeros((L,), jnp.float32) + (jnp.sum(e0) + jnp.sum(e1))
            e0 = e0 / sv
            e1 = e1 / sv

            acc = [jnp.zeros((L,), jnp.float32) for _ in range(NDC)]
            for k in range(DEG):
                w = e0[k] if k < L else e1[k - L]
                for dc in range(NDC):
                    acc[dc] = acc[dc] + w * ft_buf[b, i, k, pl.ds(dc * L, L)]
            for dc in range(NDC):
                out_buf[b, i, pl.ds(dc * L, L)] = acc[dc]
            return 0

        lax.fori_loop(0, NB, node_body, 0)

    start_in(0, 0)
    start_in(1, 1)

    def pair_body(tt, _):
        for b in (0, 1):
            t = tt * 2 + b
            blk = t * NW + wid

            @pl.when(blk < NBLK)
            def _():
                base = blk * NB
                for c in in_copies(t, b):
                    c.wait()

                @pl.when(tt >= 1)
                def _():
                    prev = P_TC + ((t - 2) * NW + wid) * NB
                    pltpu.make_async_copy(
                        out_buf.at[b], out_hbm.at[pl.ds(prev, NB)],
                        out_sems[b]).wait()

                compute_block(b)
                pltpu.make_async_copy(
                    out_buf.at[b], out_hbm.at[pl.ds(P_TC + base, NB)],
                    out_sems[b]).start()
                start_in(t + 2, b)
        return 0

    lax.fori_loop(0, TMAX // 2, pair_body, 0)

    for b in (0, 1):
        pltpu.make_async_copy(out_buf.at[b], out_hbm.at[pl.ds(0, NB)],
                              out_sems[b]).wait()


_gat_sc = functools.partial(
    pl.kernel,
    out_type=jax.ShapeDtypeStruct((N, D), jnp.float32),
    mesh=_mesh,
    compiler_params=pltpu.CompilerParams(needs_layout_passes=False,
                                         use_tc_tiling_on_sc=False),
    scratch_types=[
        pltpu.VMEM((2, NB, DEG, D), jnp.float32),
        pltpu.VMEM((2, DEG, NB), jnp.float32),
        pltpu.VMEM((2, NB), jnp.float32),
        pltpu.VMEM((2, NB, D), jnp.float32),
        pltpu.SemaphoreType.DMA,
        pltpu.SemaphoreType.DMA,
        pltpu.SemaphoreType.DMA,
        pltpu.SemaphoreType.DMA,
    ],
)(_gat_sc_body)


def _gat_tc_block(a1_ref, a2_ref, ft_ref, out_ref):
    i = pl.program_id(0)
    a1 = a1_ref[pl.ds(i * BLK_TC, BLK_TC)]
    a2 = a2_ref[...]
    ft = ft_ref[...]
    a = a1[None, :] + a2
    l = jnp.where(a > 0, a, 0.01 * a)
    m = jnp.max(l, axis=0, keepdims=True)
    e = jnp.exp(l - m)
    wk = e / jnp.sum(e, axis=0, keepdims=True)
    w = wk.T
    out_ref[...] = jnp.sum(w[:, :, None] * ft, axis=1)


def _gat_tc(a1f, a2km, ft):
    return pl.pallas_call(
        _gat_tc_block,
        grid=(P_TC // BLK_TC,),
        in_specs=[
            pl.BlockSpec((N,), lambda i: (0,)),
            pl.BlockSpec((DEG, BLK_TC), lambda i: (0, i)),
            pl.BlockSpec((BLK_TC, DEG, D), lambda i: (i, 0, 0)),
        ],
        out_specs=pl.BlockSpec((BLK_TC, D), lambda i: (i, 0)),
        out_shape=jax.ShapeDtypeStruct((P_TC, D), jnp.float32),
    )(a1f, a2km, ft)


@jax.jit
def kernel(a1, a2, ft):
    a1f = a1.reshape(N)
    a2km = a2.reshape(N, DEG).T
    out_sc = _gat_sc(a1f, a2km, ft)
    out_tc = _gat_tc(a1f, a2km, ft)
    return lax.dynamic_update_slice(out_sc, out_tc, (0, 0))
